# baseline (speedup 1.0000x reference)
import functools

import jax
import jax.numpy as jnp
from jax import lax
from jax.experimental import pallas as pl
from jax.experimental.pallas import tpu as pltpu
from jax.experimental.pallas import tpu_sc as plsc

_B = 16384
_D = 128
_V = 1000


_B_SC = 12288

_info = plsc.get_sparse_core_info()
_NC, _NS = _info.num_cores, _info.num_subcores
_NW = _NC * _NS
_BPW = _B_SC // _NW
_NCHUNK = 4
_C = _BPW // _NCHUNK

_VPT = 64
_VREM = _V - _VPT * (_NS - 1)


@functools.partial(
    pl.kernel,
    mesh=plsc.VectorSubcoreMesh(core_axis_name="c", subcore_axis_name="s"),
    out_type=jax.ShapeDtypeStruct((_B_SC, _D), jnp.float32),
    scratch_types=[
        pltpu.VMEM((_BPW,), jnp.int32),
        pltpu.VMEM((_NCHUNK, _C, _D), jnp.float32),
        pltpu.VMEM_SHARED((_V, _D), jnp.float32),
        pltpu.SemaphoreType.DMA,
        pltpu.SemaphoreType.DMA,
        pltpu.SemaphoreType.DMA,
    ],
)
def _sc_gather(idx_hbm, table_hbm, out_hbm, idx_v, rows_v, tbl_s, gsem, ssem, tsem):
    cid = lax.axis_index("c")
    sid = lax.axis_index("s")
    wid = sid * _NC + cid
    base = wid * _BPW
    with jax.named_scope("stage_table"):
        icopy = pltpu.async_copy(idx_hbm.at[pl.ds(base, _BPW)], idx_v, tsem)

        @pl.when(sid < _NS - 1)
        def _():
            pltpu.sync_copy(
                table_hbm.at[pl.ds(sid * _VPT, _VPT)],
                tbl_s.at[pl.ds(sid * _VPT, _VPT)],
            )

        @pl.when(sid == _NS - 1)
        def _():
            pltpu.sync_copy(
                table_hbm.at[pl.ds(_VPT * (_NS - 1), _VREM)],
                tbl_s.at[pl.ds(_VPT * (_NS - 1), _VREM)],
            )
    with jax.named_scope("stage_wait"):
        icopy.wait()
        plsc.subcore_barrier()
    with jax.named_scope("gather_store"):
        gathers = []
        for k in range(_NCHUNK):
            gathers.append(
                pltpu.async_copy(
                    tbl_s.at[idx_v.at[pl.ds(k * _C, _C)]], rows_v.at[k], gsem
                )
            )
        stores = []
        for k in range(_NCHUNK):
            gathers[k].wait()
            stores.append(
                pltpu.async_copy(
                    rows_v.at[k], out_hbm.at[pl.ds(base + k * _C, _C)], ssem
                )
            )
    with jax.named_scope("store_drain"):
        for k in range(_NCHUNK):
            stores[k].wait()


_B_TC = _B - _B_SC
_R = 512
_VP = 1024


def _tc_body(t_ref, w_ref, o_ref):
    tb = t_ref[...]
    oh = (tb == lax.broadcasted_iota(jnp.int32, (_R, _VP), 1)).astype(jnp.float32)
    o_ref[...] = jnp.dot(
        oh, w_ref[...], preferred_element_type=jnp.float32,
        precision=jax.lax.Precision.HIGHEST,
    )


_tc_gather = pl.pallas_call(
    _tc_body,
    grid=(_B_TC // _R,),
    in_specs=[
        pl.BlockSpec((_R, 1), lambda i: (i, 0)),
        pl.BlockSpec((_VP, _D), lambda i: (0, 0)),
    ],
    out_specs=pl.BlockSpec((_R, _D), lambda i: (i, 0)),
    out_shape=jax.ShapeDtypeStruct((_B_TC, _D), jnp.float32),
)


@jax.jit
def kernel(t, W):
    w_pad = jnp.pad(W, ((0, _VP - _V), (0, 0)))
    sc_part = _sc_gather(t[:_B_SC], W)
    tc_part = _tc_gather(t[_B_SC:].reshape(-1, 1), w_pad)
    return jnp.concatenate([sc_part, tc_part], axis=0)


# speedup vs baseline: 1.8435x; 1.8435x over previous
import functools

import jax
import jax.numpy as jnp
from jax import lax
from jax.experimental import pallas as pl
from jax.experimental.pallas import tpu as pltpu
from jax.experimental.pallas import tpu_sc as plsc

_B = 16384
_D = 128
_V = 1000

_info = plsc.get_sparse_core_info()
_NC, _NS = _info.num_cores, _info.num_subcores
_NW = _NC * _NS
_BPW = _B // _NW
_NCHUNK = 8
_C = _BPW // _NCHUNK

_VPT = 64
_VREM = _V - _VPT * (_NS - 1)


@functools.partial(
    pl.kernel,
    mesh=plsc.VectorSubcoreMesh(core_axis_name="c", subcore_axis_name="s"),
    out_type=jax.ShapeDtypeStruct((_B, _D), jnp.float32),
    scratch_types=[
        pltpu.VMEM((_BPW,), jnp.int32),
        pltpu.VMEM((_NCHUNK, _C, _D), jnp.float32),
        pltpu.VMEM_SHARED((_V, _D), jnp.float32),
        pltpu.SemaphoreType.DMA,
        pltpu.SemaphoreType.DMA,
        pltpu.SemaphoreType.DMA,
    ],
)
def _sc_gather(idx_hbm, table_hbm, out_hbm, idx_v, rows_v, tbl_s, gsem, ssem, tsem):
    cid = lax.axis_index("c")
    sid = lax.axis_index("s")
    wid = sid * _NC + cid
    base = wid * _BPW
    with jax.named_scope("stage_table"):
        icopy = pltpu.async_copy(idx_hbm.at[pl.ds(base, _BPW)], idx_v, tsem)

        @pl.when(sid < _NS - 1)
        def _():
            pltpu.sync_copy(
                table_hbm.at[pl.ds(sid * _VPT, _VPT)],
                tbl_s.at[pl.ds(sid * _VPT, _VPT)],
            )

        @pl.when(sid == _NS - 1)
        def _():
            pltpu.sync_copy(
                table_hbm.at[pl.ds(_VPT * (_NS - 1), _VREM)],
                tbl_s.at[pl.ds(_VPT * (_NS - 1), _VREM)],
            )
    with jax.named_scope("stage_wait"):
        icopy.wait()
        plsc.subcore_barrier()
    with jax.named_scope("gather_store"):
        gathers = []
        for k in range(_NCHUNK):
            gathers.append(
                pltpu.async_copy(
                    tbl_s.at[idx_v.at[pl.ds(k * _C, _C)]], rows_v.at[k], gsem
                )
            )
        stores = []
        for k in range(_NCHUNK):
            gathers[k].wait()
            stores.append(
                pltpu.async_copy(
                    rows_v.at[k], out_hbm.at[pl.ds(base + k * _C, _C)], ssem
                )
            )
    with jax.named_scope("store_drain"):
        for k in range(_NCHUNK):
            stores[k].wait()


@jax.jit
def kernel(t, W):
    return _sc_gather(t, W)
